# Initial kernel scaffold; baseline (speedup 1.0000x reference)
import jax
import jax.numpy as jnp
from jax.experimental import pallas as pl


def kernel(p_bboxs_xywh, g_bboxs_ltrb, p_labels, g_labels, p_keypoints, g_keypoints, anc):
    raise NotImplementedError("write your pallas kernel here")


# speedup vs baseline: 15.3928x; 15.3928x over previous
import functools

import jax
import jax.numpy as jnp
from jax.experimental import pallas as pl

_B = 32
_N = 16800
_NEG_RATIO = 3
_EPS = float(jnp.finfo(jnp.float32).eps)


def _smooth_l1(p, t):
    d = p - t
    ad = jnp.abs(d)
    return jnp.where(ad < 1.0, 0.5 * d * d, ad - 0.5)


def _phase1_kernel(gb_ref, pb_ref, plab_ref, glab_ref, pk_ref, gk_ref, anc_ref,
                   v_ref, ll_ref, lb_ref, lk_ref, llp_ref, pn_ref):
    ax = anc_ref[0]
    ay = anc_ref[1]
    aw = anc_ref[2]
    ah = anc_ref[3]
    inv_sx = 1.0 / (0.1 * aw)
    inv_sy = 1.0 / (0.1 * ah)

    glab = glab_ref[...]
    mask_pos = glab > 0
    maskf = mask_pos.astype(jnp.float32)
    pn_ref[...] = jnp.sum(mask_pos.astype(jnp.int32), axis=1, keepdims=True)

    gl = gb_ref[0]
    gt = gb_ref[1]
    gr = gb_ref[2]
    gbm = gb_ref[3]
    g_cx = (gl + gr) * 0.5
    g_cy = (gt + gbm) * 0.5
    g_w = gr - gl
    g_h = gbm - gt
    dx = (g_cx - ax) * inv_sx
    dy = (g_cy - ay) * inv_sy
    dw = jnp.log(jnp.maximum(g_w / aw, 1e-8)) * 5.0
    dh = jnp.log(jnp.maximum(g_h / ah, 1e-8)) * 5.0
    lb = (_smooth_l1(pb_ref[0], dx) + _smooth_l1(pb_ref[1], dy)
          + _smooth_l1(pb_ref[2], dw) + _smooth_l1(pb_ref[3], dh))
    lb_ref[...] = jnp.sum(maskf * lb, axis=1, keepdims=True)

    lk = jnp.zeros_like(lb)
    kmask = mask_pos
    for j in range(5):
        gx = gk_ref[2 * j]
        gy = gk_ref[2 * j + 1]
        kmask = kmask & (gx > 0) & (gy > 0)
        lk = lk + _smooth_l1(pk_ref[2 * j], (gx - ax) * inv_sx)
        lk = lk + _smooth_l1(pk_ref[2 * j + 1], (gy - ay) * inv_sy)
    lk_ref[...] = jnp.sum(kmask.astype(jnp.float32) * lk, axis=1, keepdims=True)

    p0 = plab_ref[0]
    p1 = plab_ref[1]
    m = jnp.maximum(p0, p1)
    lse = m + jnp.log(jnp.exp(p0 - m) + jnp.exp(p1 - m))
    sel = jnp.where(glab == 1, p1, p0)
    ll = lse - sel
    ll_ref[...] = ll
    v_ref[...] = jnp.where(mask_pos, 0.0, ll)
    llp_ref[...] = jnp.sum(maskf * ll, axis=1, keepdims=True)


def _phase2_kernel(v_ref, ll_ref, lb_ref, lk_ref, llp_ref, pn_ref, out_ref):
    v = v_ref[...]
    ll = ll_ref[...]
    v_int = jax.lax.bitcast_convert_type(v, jnp.int32)
    pos_num = pn_ref[...]
    k = jnp.minimum(_NEG_RATIO * pos_num, _N)

    inf_bits = jnp.int32(0x7F800000)

    def val_step(_, carry):
        lo, hi = carry
        mid = lo + (hi - lo) // 2
        cnt = jnp.sum((v_int > mid).astype(jnp.int32), axis=1, keepdims=True)
        pred = cnt < k
        return jnp.where(pred, lo, mid + 1), jnp.where(pred, mid, hi)

    lo0 = jnp.zeros_like(k)
    hi0 = jnp.full_like(k, inf_bits)
    t_int, _ = jax.lax.fori_loop(0, 31, val_step, (lo0, hi0))

    gt_mask = v_int > t_int
    c_gt = jnp.sum(gt_mask.astype(jnp.int32), axis=1, keepdims=True)
    s_gt = jnp.sum(jnp.where(gt_mask, v, 0.0), axis=1, keepdims=True)
    k_tie = k - c_gt
    t_f = jax.lax.bitcast_convert_type(t_int, jnp.float32)

    z = v == 0.0
    idx = jax.lax.broadcasted_iota(jnp.int32, v.shape, 1)

    def idx_step(_, carry):
        lo, hi = carry
        mid = lo + (hi - lo) // 2
        cnt = jnp.sum((z & (idx <= mid)).astype(jnp.int32), axis=1, keepdims=True)
        pred = cnt >= k_tie
        return jnp.where(pred, lo, mid + 1), jnp.where(pred, mid, hi)

    ilo0 = jnp.zeros_like(k)
    ihi0 = jnp.full_like(k, _N - 1)
    i_star, _ = jax.lax.fori_loop(0, 15, idx_step, (ilo0, ihi0))
    zsel = z & (idx <= i_star)
    contrib_zero = jnp.sum(jnp.where(zsel, ll, 0.0), axis=1, keepdims=True)

    contrib_tie = jnp.where(t_int > 0, k_tie.astype(jnp.float32) * t_f,
                            contrib_zero)
    neg = jnp.where(k > 0, s_gt + contrib_tie, 0.0)

    loss_labels = llp_ref[...] + neg
    pos_f = pos_num.astype(jnp.float32)
    num_mask = (pos_num > 0).astype(jnp.float32)
    denom = jnp.maximum(pos_f, _EPS)
    per = (lb_ref[...] + loss_labels + lk_ref[...]) * num_mask / denom
    out_ref[...] = jnp.sum(per, keepdims=True) * (1.0 / _B)


@jax.jit
def kernel(p_bboxs_xywh, g_bboxs_ltrb, p_labels, g_labels, p_keypoints,
           g_keypoints, anc):
    gb = jnp.transpose(g_bboxs_ltrb, (2, 0, 1))
    pb = jnp.transpose(p_bboxs_xywh, (2, 0, 1))
    plab = jnp.transpose(p_labels, (2, 0, 1))
    pk = jnp.transpose(p_keypoints, (2, 0, 1))
    gk = jnp.transpose(g_keypoints, (2, 0, 1))
    anc_t = jnp.transpose(anc, (2, 0, 1))
    glab = g_labels.astype(jnp.int32)

    bc = 8
    grid = (_B // bc,)

    def bmap3(d):
        return pl.BlockSpec((d, bc, _N), lambda i: (0, i, 0))

    v, ll, lb, lk, llp, pn = pl.pallas_call(
        _phase1_kernel,
        grid=grid,
        in_specs=[
            bmap3(4), bmap3(4), bmap3(2),
            pl.BlockSpec((bc, _N), lambda i: (i, 0)),
            bmap3(10), bmap3(10),
            pl.BlockSpec((4, 1, _N), lambda i: (0, 0, 0)),
        ],
        out_specs=[
            pl.BlockSpec((bc, _N), lambda i: (i, 0)),
            pl.BlockSpec((bc, _N), lambda i: (i, 0)),
            pl.BlockSpec((bc, 1), lambda i: (i, 0)),
            pl.BlockSpec((bc, 1), lambda i: (i, 0)),
            pl.BlockSpec((bc, 1), lambda i: (i, 0)),
            pl.BlockSpec((bc, 1), lambda i: (i, 0)),
        ],
        out_shape=[
            jax.ShapeDtypeStruct((_B, _N), jnp.float32),
            jax.ShapeDtypeStruct((_B, _N), jnp.float32),
            jax.ShapeDtypeStruct((_B, 1), jnp.float32),
            jax.ShapeDtypeStruct((_B, 1), jnp.float32),
            jax.ShapeDtypeStruct((_B, 1), jnp.float32),
            jax.ShapeDtypeStruct((_B, 1), jnp.int32),
        ],
    )(gb, pb, plab, glab, pk, gk, anc_t)

    out = pl.pallas_call(
        _phase2_kernel,
        out_shape=jax.ShapeDtypeStruct((1, 1), jnp.float32),
    )(v, ll, lb, lk, llp, pn)
    return out[0, 0]
